# Initial kernel scaffold; baseline (speedup 1.0000x reference)
import jax
import jax.numpy as jnp
from jax.experimental import pallas as pl


def kernel(x, edge_index, W, b):
    raise NotImplementedError("write your pallas kernel here")


# speedup vs baseline: 3.7996x; 3.7996x over previous
import jax
import jax.numpy as jnp
from jax import lax
from jax.experimental import pallas as pl
from jax.experimental.pallas import tpu as pltpu
from jax.experimental.pallas import tpu_sc as plsc

N_NODES = 10000
N_EDGES = 320000
D_FEAT = 128
AUG_PROB = 0.2

NUM_CORES = 2
NUM_SUBCORES = 16
LANES = 16

CHUNK = 128
SUB = 8
LOADS_PER_TILE = 20
ROWS_PER_TILE = LOADS_PER_TILE * SUB
E_PAD = NUM_SUBCORES * ROWS_PER_TILE * 128
ACC_ROWS = 10240
STRIPE = ACC_ROWS // NUM_SUBCORES
DUMMY = N_NODES


def _prologue_body(x_ref, uf1_ref, uf2_ref, ue1_ref, ue2_ref, dst_ref,
                   x1_ref, x2_ref, d1_ref, d2_ref):
    x = x_ref[...]
    x1_ref[...] = x * (uf1_ref[...] > AUG_PROB).astype(jnp.float32)
    x2_ref[...] = x * (uf2_ref[...] > AUG_PROB).astype(jnp.float32)
    dst = dst_ref[...]
    pad = jnp.full(((E_PAD - N_EDGES) // 128, 128), DUMMY, jnp.int32)
    d1 = jnp.where(ue1_ref[...] > AUG_PROB, dst, DUMMY)
    d2 = jnp.where(ue2_ref[...] > AUG_PROB, dst, DUMMY)
    d1_ref[...] = jnp.concatenate([d1, pad], axis=0)
    d2_ref[...] = jnp.concatenate([d2, pad], axis=0)


def _prologue(x, uf1, uf2, ue1, ue2, dst):
    return pl.pallas_call(
        _prologue_body,
        out_shape=(
            jax.ShapeDtypeStruct((N_NODES, D_FEAT), jnp.float32),
            jax.ShapeDtypeStruct((N_NODES, D_FEAT), jnp.float32),
            jax.ShapeDtypeStruct((E_PAD // 128, 128), jnp.int32),
            jax.ShapeDtypeStruct((E_PAD // 128, 128), jnp.int32),
        ),
    )(x, uf1, uf2, ue1, ue2, dst)


def _sc_body(src_hbm, d1_hbm, d2_hbm, x1_hbm, x2_hbm,
             acc1_out, deg1_out, acc2_out, deg2_out,
             src_v, dst_v, rows_v, ones_v, zrow_v, acc_sh, sem):
    c = lax.axis_index("c")
    s = lax.axis_index("s")

    zeros16 = jnp.zeros((LANES,), jnp.float32)
    for r in range(16):
        for j in range(D_FEAT // LANES):
            zrow_v[r, pl.ds(LANES * j, LANES)] = zeros16

    def fill_ones(i, _):
        for j in range(D_FEAT // LANES):
            ones_v[pl.ds(i, 1), pl.ds(LANES * j, LANES)] = jnp.ones(
                (1, LANES), jnp.float32)
        return 0
    lax.fori_loop(0, CHUNK, fill_ones, 0)

    def zero_acc():
        def zero_step(j, _):
            pltpu.sync_copy(zrow_v, acc_sh.at[pl.ds(s * STRIPE + j * 16, 16)])
            return 0
        lax.fori_loop(0, STRIPE // 16, zero_step, 0)

    def scatter_pass(x_hbm, dst_hbm, gather_rows):
        def load_step(i, _):
            r8 = s * ROWS_PER_TILE + i * SUB
            pltpu.sync_copy(dst_hbm.at[pl.ds(r8, SUB)], dst_v)
            if gather_rows:
                pltpu.sync_copy(src_hbm.at[pl.ds(r8, SUB)], src_v)
            for j in range(SUB):
                if gather_rows:
                    pltpu.async_copy(
                        x_hbm.at[src_v.at[j]], rows_v, sem).wait()
                    pltpu.sync_copy(
                        rows_v, acc_sh.at[dst_v.at[j]], add=True)
                else:
                    pltpu.sync_copy(
                        ones_v, acc_sh.at[dst_v.at[j]], add=True)
            return 0
        lax.fori_loop(0, LOADS_PER_TILE, load_step, 0)

    def copyout(out_hbm):
        def out_step(j, _):
            row = s * STRIPE + j * CHUNK
            pltpu.sync_copy(acc_sh.at[pl.ds(row, CHUNK)], rows_v)
            pltpu.sync_copy(rows_v, out_hbm.at[pl.ds(row, CHUNK)])
            return 0
        lax.fori_loop(0, STRIPE // CHUNK, out_step, 0)

    def phase(fn1, fn2):
        pl.when(c == 0)(fn1)
        pl.when(c == 1)(fn2)
        plsc.subcore_barrier()

    phase(zero_acc, zero_acc)
    phase(lambda: scatter_pass(x1_hbm, d1_hbm, True),
          lambda: scatter_pass(x2_hbm, d2_hbm, True))
    phase(lambda: copyout(acc1_out), lambda: copyout(acc2_out))
    phase(zero_acc, zero_acc)
    phase(lambda: scatter_pass(x1_hbm, d1_hbm, False),
          lambda: scatter_pass(x2_hbm, d2_hbm, False))
    phase(lambda: copyout(deg1_out), lambda: copyout(deg2_out))


def _sc_aggregate(src_pad, d1, d2, x1, x2):
    mesh = plsc.VectorSubcoreMesh(
        core_axis_name="c", subcore_axis_name="s",
        num_cores=NUM_CORES, num_subcores=NUM_SUBCORES)
    f32 = jnp.float32
    kern = pl.kernel(
        _sc_body,
        out_type=(
            jax.ShapeDtypeStruct((ACC_ROWS, D_FEAT), f32),
            jax.ShapeDtypeStruct((ACC_ROWS, D_FEAT), f32),
            jax.ShapeDtypeStruct((ACC_ROWS, D_FEAT), f32),
            jax.ShapeDtypeStruct((ACC_ROWS, D_FEAT), f32),
        ),
        mesh=mesh,
        scratch_types=[
            pltpu.VMEM((SUB, 128), jnp.int32),
            pltpu.VMEM((SUB, 128), jnp.int32),
            pltpu.VMEM((CHUNK, D_FEAT), f32),
            pltpu.VMEM((CHUNK, D_FEAT), f32),
            pltpu.VMEM((16, D_FEAT), f32),
            pltpu.VMEM_SHARED((ACC_ROWS, D_FEAT), f32),
            pltpu.SemaphoreType.DMA,
        ],
    )
    return kern(src_pad, d1, d2, x1, x2)


def _epilogue_body(acc1_ref, deg1_ref, x1_ref, acc2_ref, deg2_ref, x2_ref,
                   w_ref, b_ref, z1_ref, z2_ref):
    w = w_ref[...]
    b = b_ref[...]

    def one(acc_ref, deg_ref, x_ref, z_ref):
        deg = jnp.maximum(deg_ref[...][:, 0:1], 1.0)
        h = acc_ref[...] / deg + x_ref[...]
        z_ref[...] = jnp.dot(h, w, preferred_element_type=jnp.float32,
                             precision=lax.Precision.HIGHEST) + b

    one(acc1_ref, deg1_ref, x1_ref, z1_ref)
    one(acc2_ref, deg2_ref, x2_ref, z2_ref)


def _epilogue(acc1, deg1, x1, acc2, deg2, x2, W, b2d):
    blk = 1000
    grid = (N_NODES // blk,)
    row_spec = pl.BlockSpec((blk, D_FEAT), lambda i: (i, 0))
    w_spec = pl.BlockSpec((D_FEAT, D_FEAT), lambda i: (0, 0))
    b_spec = pl.BlockSpec((1, D_FEAT), lambda i: (0, 0))
    return pl.pallas_call(
        _epilogue_body,
        grid=grid,
        in_specs=[row_spec, row_spec, row_spec,
                  row_spec, row_spec, row_spec, w_spec, b_spec],
        out_specs=(row_spec, row_spec),
        out_shape=(
            jax.ShapeDtypeStruct((N_NODES, D_FEAT), jnp.float32),
            jax.ShapeDtypeStruct((N_NODES, D_FEAT), jnp.float32),
        ),
    )(acc1, deg1, x1, acc2, deg2, x2, W, b2d)


@jax.jit
def kernel(x, edge_index, W, b):
    key = jax.random.key(42)
    k1, k2 = jax.random.split(key)
    ke1, kf1 = jax.random.split(k1)
    ke2, kf2 = jax.random.split(k2)
    ue1 = jax.random.uniform(ke1, (N_EDGES,))
    uf1 = jax.random.uniform(kf1, x.shape)
    ue2 = jax.random.uniform(ke2, (N_EDGES,))
    uf2 = jax.random.uniform(kf2, x.shape)

    src = edge_index[0]
    dst = edge_index[1]

    x1, x2, d1, d2 = _prologue(
        x, uf1, uf2,
        ue1.reshape(N_EDGES // 128, 128),
        ue2.reshape(N_EDGES // 128, 128),
        dst.reshape(N_EDGES // 128, 128),
    )

    src_pad = jnp.concatenate(
        [src, jnp.zeros((E_PAD - N_EDGES,), jnp.int32)]).reshape(
            E_PAD // 128, 128)
    acc1, deg1, acc2, deg2 = _sc_aggregate(src_pad, d1, d2, x1, x2)

    z1, z2 = _epilogue(
        acc1[:N_NODES], deg1[:N_NODES], x1,
        acc2[:N_NODES], deg2[:N_NODES], x2,
        W, b.reshape(1, D_FEAT))
    return (z1, z2)


# speedup vs baseline: 4.3623x; 1.1481x over previous
import dataclasses

import jax
import jax.numpy as jnp
from jax import lax
from jax.experimental import pallas as pl
from jax.experimental.pallas import tpu as pltpu
from jax.experimental.pallas import tpu_sc as plsc

N_NODES = 10000
N_EDGES = 320000
D_FEAT = 128
AUG_PROB = 0.2

NUM_CORES = 2
NUM_SUBCORES = 16
LANES = 16

CHUNK = 128
SUB = 8
LOADS_PER_TILE = 20
ROWS_PER_TILE = LOADS_PER_TILE * SUB
E_PAD = NUM_SUBCORES * ROWS_PER_TILE * 128
ACC_ROWS = 10240
STRIPE = ACC_ROWS // NUM_SUBCORES
DUMMY = N_NODES


def _prologue_body(x_ref, uf1_ref, uf2_ref, ue1_ref, ue2_ref, dst_ref,
                   x1_ref, x2_ref, d1_ref, d2_ref):
    x = x_ref[...]
    xpad = jnp.zeros((ACC_ROWS - N_NODES, D_FEAT), jnp.float32)
    x1 = x * (uf1_ref[...] > AUG_PROB).astype(jnp.float32)
    x2 = x * (uf2_ref[...] > AUG_PROB).astype(jnp.float32)
    x1_ref[...] = jnp.concatenate([x1, xpad], axis=0)
    x2_ref[...] = jnp.concatenate([x2, xpad], axis=0)
    dst = dst_ref[...]
    pad = jnp.full(((E_PAD - N_EDGES) // 128, 128), DUMMY, jnp.int32)
    d1 = jnp.where(ue1_ref[...] > AUG_PROB, dst, DUMMY)
    d2 = jnp.where(ue2_ref[...] > AUG_PROB, dst, DUMMY)
    d1_ref[...] = jnp.concatenate([d1, pad], axis=0)
    d2_ref[...] = jnp.concatenate([d2, pad], axis=0)


def _prologue(x, uf1, uf2, ue1, ue2, dst):
    return pl.pallas_call(
        _prologue_body,
        out_shape=(
            jax.ShapeDtypeStruct((ACC_ROWS, D_FEAT), jnp.float32),
            jax.ShapeDtypeStruct((ACC_ROWS, D_FEAT), jnp.float32),
            jax.ShapeDtypeStruct((E_PAD // 128, 128), jnp.int32),
            jax.ShapeDtypeStruct((E_PAD // 128, 128), jnp.int32),
        ),
    )(x, uf1, uf2, ue1, ue2, dst)


def _sc_body(src_hbm, d1_hbm, d2_hbm, x1_hbm, x2_hbm,
             acc1_out, deg1_out, acc2_out, deg2_out,
             src_v, dst_v, rows_v, hist_v, zrow_v, acc_sh, sem):
    c = lax.axis_index("c")
    s = lax.axis_index("s")

    zeros16 = jnp.zeros((LANES,), jnp.float32)
    for r in range(16):
        for j in range(D_FEAT // LANES):
            zrow_v[r, pl.ds(LANES * j, LANES)] = zeros16

    def fill_hist(i, _):
        for j in range(D_FEAT // LANES):
            hist_v[i, pl.ds(LANES * j, LANES)] = zeros16
        return 0
    lax.fori_loop(0, ACC_ROWS // 128, fill_hist, 0)

    def zero_acc():
        def zero_step(j, _):
            pltpu.sync_copy(zrow_v, acc_sh.at[pl.ds(s * STRIPE + j * 16, 16)])
            return 0
        lax.fori_loop(0, STRIPE // 16, zero_step, 0)

    ones16 = jnp.ones((LANES,), jnp.float32)

    def scatter_pass(x_hbm, dst_hbm):
        def load_step(i, _):
            r8 = s * ROWS_PER_TILE + i * SUB
            pltpu.sync_copy(dst_hbm.at[pl.ds(r8, SUB)], dst_v)
            pltpu.sync_copy(src_hbm.at[pl.ds(r8, SUB)], src_v)
            for j in range(SUB):
                pltpu.async_copy(x_hbm.at[src_v.at[j]], rows_v, sem).wait()
                pltpu.sync_copy(rows_v, acc_sh.at[dst_v.at[j]], add=True)
                for k in range(CHUNK // LANES):
                    dv = dst_v[j, pl.ds(LANES * k, LANES)]
                    plsc.addupdate_scatter(
                        hist_v,
                        [lax.shift_right_logical(dv, 7),
                         lax.bitwise_and(dv, 127)],
                        ones16)
            return 0
        lax.fori_loop(0, LOADS_PER_TILE, load_step, 0)

    def copyout(out_hbm):
        def out_step(j, _):
            row = s * STRIPE + j * CHUNK
            pltpu.sync_copy(acc_sh.at[pl.ds(row, CHUNK)], rows_v)
            pltpu.sync_copy(rows_v, out_hbm.at[pl.ds(row, CHUNK)])
            return 0
        lax.fori_loop(0, STRIPE // CHUNK, out_step, 0)

    def phase(fn1, fn2):
        pl.when(c == 0)(fn1)
        pl.when(c == 1)(fn2)
        plsc.subcore_barrier()

    def finish(acc_out, deg_out):
        copyout(acc_out)
        pltpu.sync_copy(hist_v, deg_out.at[s])

    phase(zero_acc, zero_acc)
    phase(lambda: scatter_pass(x1_hbm, d1_hbm),
          lambda: scatter_pass(x2_hbm, d2_hbm))
    phase(lambda: finish(acc1_out, deg1_out),
          lambda: finish(acc2_out, deg2_out))


def _sc_aggregate(src_pad, d1, d2, x1, x2):
    mesh = plsc.VectorSubcoreMesh(
        core_axis_name="c", subcore_axis_name="s",
        num_cores=NUM_CORES, num_subcores=NUM_SUBCORES)
    f32 = jnp.float32
    cp = pltpu.CompilerParams()
    if "needs_layout_passes" in pltpu.CompilerParams.__dataclass_fields__:
        cp = dataclasses.replace(cp, needs_layout_passes=False)
    kern = pl.kernel(
        _sc_body,
        out_type=(
            jax.ShapeDtypeStruct((ACC_ROWS, D_FEAT), f32),
            jax.ShapeDtypeStruct((NUM_SUBCORES, ACC_ROWS // 128, 128), f32),
            jax.ShapeDtypeStruct((ACC_ROWS, D_FEAT), f32),
            jax.ShapeDtypeStruct((NUM_SUBCORES, ACC_ROWS // 128, 128), f32),
        ),
        mesh=mesh,
        scratch_types=[
            pltpu.VMEM((SUB, 128), jnp.int32),
            pltpu.VMEM((SUB, 128), jnp.int32),
            pltpu.VMEM((CHUNK, D_FEAT), f32),
            pltpu.VMEM((ACC_ROWS // 128, 128), f32),
            pltpu.VMEM((16, D_FEAT), f32),
            pltpu.VMEM_SHARED((ACC_ROWS, D_FEAT), f32),
            pltpu.SemaphoreType.DMA,
        ],
        compiler_params=cp,
    )
    return kern(src_pad, d1, d2, x1, x2)


def _epilogue_body(acc1_ref, deg1_ref, x1_ref, acc2_ref, deg2_ref, x2_ref,
                   w_ref, b_ref, z1_ref, z2_ref):
    w = w_ref[...]
    b = b_ref[...]
    blk = acc1_ref.shape[0]
    rows8 = blk // 128
    rsel = (lax.broadcasted_iota(jnp.int32, (blk, rows8), 0) // 128 ==
            lax.broadcasted_iota(jnp.int32, (blk, rows8), 1)).astype(
                jnp.float32)
    csel = (lax.broadcasted_iota(jnp.int32, (blk, 128), 0) % 128 ==
            lax.broadcasted_iota(jnp.int32, (blk, 128), 1)).astype(
                jnp.float32)

    def one(acc_ref, deg_ref, x_ref, z_ref):
        grid = jnp.sum(deg_ref[...], axis=0)
        t = jnp.dot(rsel, grid, preferred_element_type=jnp.float32,
                    precision=lax.Precision.HIGHEST)
        deg = jnp.sum(t * csel, axis=1, keepdims=True)
        deg = jnp.maximum(deg, 1.0)
        h = acc_ref[...] / deg + x_ref[...]
        z_ref[...] = jnp.dot(h, w, preferred_element_type=jnp.float32,
                             precision=lax.Precision.HIGHEST) + b

    one(acc1_ref, deg1_ref, x1_ref, z1_ref)
    one(acc2_ref, deg2_ref, x2_ref, z2_ref)


def _epilogue(acc1, deg1, x1, acc2, deg2, x2, W, b2d):
    blk = 1024
    grid = (ACC_ROWS // blk,)
    row_spec = pl.BlockSpec((blk, D_FEAT), lambda i: (i, 0))
    deg_spec = pl.BlockSpec((NUM_SUBCORES, blk // 128, 128),
                            lambda i: (0, i, 0))
    w_spec = pl.BlockSpec((D_FEAT, D_FEAT), lambda i: (0, 0))
    b_spec = pl.BlockSpec((1, D_FEAT), lambda i: (0, 0))
    return pl.pallas_call(
        _epilogue_body,
        grid=grid,
        in_specs=[row_spec, deg_spec, row_spec,
                  row_spec, deg_spec, row_spec, w_spec, b_spec],
        out_specs=(row_spec, row_spec),
        out_shape=(
            jax.ShapeDtypeStruct((ACC_ROWS, D_FEAT), jnp.float32),
            jax.ShapeDtypeStruct((ACC_ROWS, D_FEAT), jnp.float32),
        ),
    )(acc1, deg1, x1, acc2, deg2, x2, W, b2d)


@jax.jit
def kernel(x, edge_index, W, b):
    key = jax.random.key(42)
    k1, k2 = jax.random.split(key)
    ke1, kf1 = jax.random.split(k1)
    ke2, kf2 = jax.random.split(k2)
    ue1 = jax.random.uniform(ke1, (N_EDGES,))
    uf1 = jax.random.uniform(kf1, x.shape)
    ue2 = jax.random.uniform(ke2, (N_EDGES,))
    uf2 = jax.random.uniform(kf2, x.shape)

    src = edge_index[0]
    dst = edge_index[1]

    x1, x2, d1, d2 = _prologue(
        x, uf1, uf2,
        ue1.reshape(N_EDGES // 128, 128),
        ue2.reshape(N_EDGES // 128, 128),
        dst.reshape(N_EDGES // 128, 128),
    )

    src_pad = jnp.concatenate(
        [src, jnp.zeros((E_PAD - N_EDGES,), jnp.int32)]).reshape(
            E_PAD // 128, 128)
    acc1, deg1, acc2, deg2 = _sc_aggregate(src_pad, d1, d2, x1, x2)

    z1, z2 = _epilogue(acc1, deg1, x1, acc2, deg2, x2,
                       W, b.reshape(1, D_FEAT))
    return (z1[:N_NODES], z2[:N_NODES])


# speedup vs baseline: 4.9761x; 1.1407x over previous
import dataclasses

import jax
import jax.numpy as jnp
from jax import lax
from jax.experimental import pallas as pl
from jax.experimental.pallas import tpu as pltpu
from jax.experimental.pallas import tpu_sc as plsc

N_NODES = 10000
N_EDGES = 320000
D_FEAT = 128
AUG_PROB = 0.2

NUM_CORES = 2
NUM_SUBCORES = 16
LANES = 16

CHUNK = 128
SUB = 8
LOADS_PER_TILE = 20
ROWS_PER_TILE = LOADS_PER_TILE * SUB
E_PAD = NUM_SUBCORES * ROWS_PER_TILE * 128
ACC_ROWS = 10240
STRIPE = ACC_ROWS // NUM_SUBCORES
DUMMY = N_NODES


def _prologue_body(x_ref, uf1_ref, uf2_ref, ue1_ref, ue2_ref, dst_ref,
                   x1_ref, x2_ref, d1_ref, d2_ref):
    x = x_ref[...]
    xpad = jnp.zeros((ACC_ROWS - N_NODES, D_FEAT), jnp.float32)
    x1 = x * (uf1_ref[...] > AUG_PROB).astype(jnp.float32)
    x2 = x * (uf2_ref[...] > AUG_PROB).astype(jnp.float32)
    x1_ref[...] = jnp.concatenate([x1, xpad], axis=0)
    x2_ref[...] = jnp.concatenate([x2, xpad], axis=0)
    dst = dst_ref[...]
    pad = jnp.full(((E_PAD - N_EDGES) // 128, 128), DUMMY, jnp.int32)
    d1 = jnp.where(ue1_ref[...] > AUG_PROB, dst, DUMMY)
    d2 = jnp.where(ue2_ref[...] > AUG_PROB, dst, DUMMY)
    d1_ref[...] = jnp.concatenate([d1, pad], axis=0)
    d2_ref[...] = jnp.concatenate([d2, pad], axis=0)


def _prologue(x, uf1, uf2, ue1, ue2, dst):
    return pl.pallas_call(
        _prologue_body,
        out_shape=(
            jax.ShapeDtypeStruct((ACC_ROWS, D_FEAT), jnp.float32),
            jax.ShapeDtypeStruct((ACC_ROWS, D_FEAT), jnp.float32),
            jax.ShapeDtypeStruct((E_PAD // 128, 128), jnp.int32),
            jax.ShapeDtypeStruct((E_PAD // 128, 128), jnp.int32),
        ),
    )(x, uf1, uf2, ue1, ue2, dst)


def _sc_body(src_hbm, d1_hbm, d2_hbm, x1_hbm, x2_hbm,
             acc1_out, deg1_out, acc2_out, deg2_out,
             src_v, dst_v, rows_v, rows2_v, hist_v, zrow_v, acc_sh,
             sem, sem2, sem3, sem4):
    c = lax.axis_index("c")
    s = lax.axis_index("s")

    zeros16 = jnp.zeros((LANES,), jnp.float32)
    for r in range(16):
        for j in range(D_FEAT // LANES):
            zrow_v[r, pl.ds(LANES * j, LANES)] = zeros16

    def fill_hist(i, _):
        for j in range(D_FEAT // LANES):
            hist_v[i, pl.ds(LANES * j, LANES)] = zeros16
        return 0
    lax.fori_loop(0, ACC_ROWS // 128, fill_hist, 0)

    def zero_acc():
        def zero_step(j, _):
            pltpu.sync_copy(zrow_v, acc_sh.at[pl.ds(s * STRIPE + j * 16, 16)])
            return 0
        lax.fori_loop(0, STRIPE // 16, zero_step, 0)

    ones16 = jnp.ones((LANES,), jnp.float32)

    def scatter_pass(x_hbm, dst_hbm):
        rows = (rows_v, rows2_v)
        gsem = (sem, sem2)
        ssem = (sem3, sem4)

        def load_step(i, _):
            r8 = s * ROWS_PER_TILE + i * SUB
            pltpu.sync_copy(dst_hbm.at[pl.ds(r8, SUB)], dst_v)
            pltpu.sync_copy(src_hbm.at[pl.ds(r8, SUB)], src_v)
            gd = [None, None]
            sd = [None, None]
            gd[0] = pltpu.async_copy(x_hbm.at[src_v.at[0]], rows[0], gsem[0])
            for j in range(SUB):
                b = j & 1
                if j + 1 < SUB:
                    b2 = (j + 1) & 1
                    if sd[b2] is not None:
                        sd[b2].wait()
                    gd[b2] = pltpu.async_copy(
                        x_hbm.at[src_v.at[j + 1]], rows[b2], gsem[b2])
                gd[b].wait()
                sd[b] = pltpu.async_copy(
                    rows[b], acc_sh.at[dst_v.at[j]], ssem[b], add=True)
                for k in range(CHUNK // LANES):
                    dv = dst_v[j, pl.ds(LANES * k, LANES)]
                    plsc.addupdate_scatter(
                        hist_v,
                        [lax.shift_right_logical(dv, 7),
                         lax.bitwise_and(dv, 127)],
                        ones16)
            sd[0].wait()
            sd[1].wait()
            return 0
        lax.fori_loop(0, LOADS_PER_TILE, load_step, 0)

    def copyout(out_hbm):
        def out_step(j, _):
            row = s * STRIPE + j * CHUNK
            pltpu.sync_copy(acc_sh.at[pl.ds(row, CHUNK)], rows_v)
            pltpu.sync_copy(rows_v, out_hbm.at[pl.ds(row, CHUNK)])
            return 0
        lax.fori_loop(0, STRIPE // CHUNK, out_step, 0)

    def phase(fn1, fn2):
        pl.when(c == 0)(fn1)
        pl.when(c == 1)(fn2)
        plsc.subcore_barrier()

    def finish(acc_out, deg_out):
        copyout(acc_out)
        pltpu.sync_copy(hist_v, deg_out.at[s])

    phase(zero_acc, zero_acc)
    phase(lambda: scatter_pass(x1_hbm, d1_hbm),
          lambda: scatter_pass(x2_hbm, d2_hbm))
    phase(lambda: finish(acc1_out, deg1_out),
          lambda: finish(acc2_out, deg2_out))


def _sc_aggregate(src_pad, d1, d2, x1, x2):
    mesh = plsc.VectorSubcoreMesh(
        core_axis_name="c", subcore_axis_name="s",
        num_cores=NUM_CORES, num_subcores=NUM_SUBCORES)
    f32 = jnp.float32
    cp = pltpu.CompilerParams()
    if "needs_layout_passes" in pltpu.CompilerParams.__dataclass_fields__:
        cp = dataclasses.replace(cp, needs_layout_passes=False)
    kern = pl.kernel(
        _sc_body,
        out_type=(
            jax.ShapeDtypeStruct((ACC_ROWS, D_FEAT), f32),
            jax.ShapeDtypeStruct((NUM_SUBCORES, ACC_ROWS // 128, 128), f32),
            jax.ShapeDtypeStruct((ACC_ROWS, D_FEAT), f32),
            jax.ShapeDtypeStruct((NUM_SUBCORES, ACC_ROWS // 128, 128), f32),
        ),
        mesh=mesh,
        scratch_types=[
            pltpu.VMEM((SUB, 128), jnp.int32),
            pltpu.VMEM((SUB, 128), jnp.int32),
            pltpu.VMEM((CHUNK, D_FEAT), f32),
            pltpu.VMEM((CHUNK, D_FEAT), f32),
            pltpu.VMEM((ACC_ROWS // 128, 128), f32),
            pltpu.VMEM((16, D_FEAT), f32),
            pltpu.VMEM_SHARED((ACC_ROWS, D_FEAT), f32),
            pltpu.SemaphoreType.DMA,
            pltpu.SemaphoreType.DMA,
            pltpu.SemaphoreType.DMA,
            pltpu.SemaphoreType.DMA,
        ],
        compiler_params=cp,
    )
    return kern(src_pad, d1, d2, x1, x2)


def _epilogue_body(acc1_ref, deg1_ref, x1_ref, acc2_ref, deg2_ref, x2_ref,
                   w_ref, b_ref, z1_ref, z2_ref):
    w = w_ref[...]
    b = b_ref[...]
    blk = acc1_ref.shape[0]
    rows8 = blk // 128
    rsel = (lax.broadcasted_iota(jnp.int32, (blk, rows8), 0) // 128 ==
            lax.broadcasted_iota(jnp.int32, (blk, rows8), 1)).astype(
                jnp.float32)
    csel = (lax.broadcasted_iota(jnp.int32, (blk, 128), 0) % 128 ==
            lax.broadcasted_iota(jnp.int32, (blk, 128), 1)).astype(
                jnp.float32)

    def one(acc_ref, deg_ref, x_ref, z_ref):
        grid = jnp.sum(deg_ref[...], axis=0)
        t = jnp.dot(rsel, grid, preferred_element_type=jnp.float32,
                    precision=lax.Precision.HIGHEST)
        deg = jnp.sum(t * csel, axis=1, keepdims=True)
        deg = jnp.maximum(deg, 1.0)
        h = acc_ref[...] / deg + x_ref[...]
        z_ref[...] = jnp.dot(h, w, preferred_element_type=jnp.float32,
                             precision=lax.Precision.HIGHEST) + b

    one(acc1_ref, deg1_ref, x1_ref, z1_ref)
    one(acc2_ref, deg2_ref, x2_ref, z2_ref)


def _epilogue(acc1, deg1, x1, acc2, deg2, x2, W, b2d):
    blk = 1024
    grid = (ACC_ROWS // blk,)
    row_spec = pl.BlockSpec((blk, D_FEAT), lambda i: (i, 0))
    deg_spec = pl.BlockSpec((NUM_SUBCORES, blk // 128, 128),
                            lambda i: (0, i, 0))
    w_spec = pl.BlockSpec((D_FEAT, D_FEAT), lambda i: (0, 0))
    b_spec = pl.BlockSpec((1, D_FEAT), lambda i: (0, 0))
    return pl.pallas_call(
        _epilogue_body,
        grid=grid,
        in_specs=[row_spec, deg_spec, row_spec,
                  row_spec, deg_spec, row_spec, w_spec, b_spec],
        out_specs=(row_spec, row_spec),
        out_shape=(
            jax.ShapeDtypeStruct((ACC_ROWS, D_FEAT), jnp.float32),
            jax.ShapeDtypeStruct((ACC_ROWS, D_FEAT), jnp.float32),
        ),
    )(acc1, deg1, x1, acc2, deg2, x2, W, b2d)


@jax.jit
def kernel(x, edge_index, W, b):
    key = jax.random.key(42)
    k1, k2 = jax.random.split(key)
    ke1, kf1 = jax.random.split(k1)
    ke2, kf2 = jax.random.split(k2)
    ue1 = jax.random.uniform(ke1, (N_EDGES,))
    uf1 = jax.random.uniform(kf1, x.shape)
    ue2 = jax.random.uniform(ke2, (N_EDGES,))
    uf2 = jax.random.uniform(kf2, x.shape)

    src = edge_index[0]
    dst = edge_index[1]

    x1, x2, d1, d2 = _prologue(
        x, uf1, uf2,
        ue1.reshape(N_EDGES // 128, 128),
        ue2.reshape(N_EDGES // 128, 128),
        dst.reshape(N_EDGES // 128, 128),
    )

    src_pad = jnp.concatenate(
        [src, jnp.zeros((E_PAD - N_EDGES,), jnp.int32)]).reshape(
            E_PAD // 128, 128)
    acc1, deg1, acc2, deg2 = _sc_aggregate(src_pad, d1, d2, x1, x2)

    z1, z2 = _epilogue(acc1, deg1, x1, acc2, deg2, x2,
                       W, b.reshape(1, D_FEAT))
    return (z1[:N_NODES], z2[:N_NODES])


# speedup vs baseline: 5.0880x; 1.0225x over previous
import dataclasses

import jax
import jax.numpy as jnp
from jax import lax
from jax.experimental import pallas as pl
from jax.experimental.pallas import tpu as pltpu
from jax.experimental.pallas import tpu_sc as plsc

N_NODES = 10000
N_EDGES = 320000
D_FEAT = 128
AUG_PROB = 0.2

NUM_CORES = 2
NUM_SUBCORES = 16
LANES = 16

CHUNK = 128
SUB = 16
LOADS_PER_TILE = 10
NBUF = 2
ROWS_PER_TILE = LOADS_PER_TILE * SUB
E_PAD = NUM_SUBCORES * ROWS_PER_TILE * 128
ACC_ROWS = 10240
STRIPE = ACC_ROWS // NUM_SUBCORES
DUMMY = N_NODES


def _prologue_body(x_ref, uf1_ref, uf2_ref, ue1_ref, ue2_ref, dst_ref,
                   x1_ref, x2_ref, d1_ref, d2_ref):
    x = x_ref[...]
    xpad = jnp.zeros((ACC_ROWS - N_NODES, D_FEAT), jnp.float32)
    x1 = x * (uf1_ref[...] > AUG_PROB).astype(jnp.float32)
    x2 = x * (uf2_ref[...] > AUG_PROB).astype(jnp.float32)
    x1_ref[...] = jnp.concatenate([x1, xpad], axis=0)
    x2_ref[...] = jnp.concatenate([x2, xpad], axis=0)
    dst = dst_ref[...]
    pad = jnp.full(((E_PAD - N_EDGES) // 128, 128), DUMMY, jnp.int32)
    d1 = jnp.where(ue1_ref[...] > AUG_PROB, dst, DUMMY)
    d2 = jnp.where(ue2_ref[...] > AUG_PROB, dst, DUMMY)
    d1_ref[...] = jnp.concatenate([d1, pad], axis=0)
    d2_ref[...] = jnp.concatenate([d2, pad], axis=0)


def _prologue(x, uf1, uf2, ue1, ue2, dst):
    return pl.pallas_call(
        _prologue_body,
        out_shape=(
            jax.ShapeDtypeStruct((ACC_ROWS, D_FEAT), jnp.float32),
            jax.ShapeDtypeStruct((ACC_ROWS, D_FEAT), jnp.float32),
            jax.ShapeDtypeStruct((E_PAD // 128, 128), jnp.int32),
            jax.ShapeDtypeStruct((E_PAD // 128, 128), jnp.int32),
        ),
    )(x, uf1, uf2, ue1, ue2, dst)


def _sc_body(src_hbm, d1_hbm, d2_hbm, x1_hbm, x2_hbm, zeros_hbm,
             acc1_out, deg1_out, acc2_out, deg2_out,
             src_v, dst_v, rows_v, rows2_v, hist_v, acc_sh,
             sem, sem2, sem3, sem4):
    c = lax.axis_index("c")
    s = lax.axis_index("s")

    zeros16 = jnp.zeros((LANES,), jnp.float32)

    def fill_hist(i, _):
        for j in range(D_FEAT // LANES):
            hist_v[i, pl.ds(LANES * j, LANES)] = zeros16
        return 0
    lax.fori_loop(0, ACC_ROWS // 128, fill_hist, 0)

    def zero_acc():
        pltpu.sync_copy(zeros_hbm.at[pl.ds(s * STRIPE, STRIPE)],
                        acc_sh.at[pl.ds(s * STRIPE, STRIPE)])

    ones16 = jnp.ones((LANES,), jnp.float32)

    def scatter_pass(x_hbm, dst_hbm):
        rows = (rows_v, rows2_v)
        gsem = (sem, sem2)
        ssem = (sem3, sem4)

        def load_step(i, _):
            r8 = s * ROWS_PER_TILE + i * SUB
            pltpu.sync_copy(dst_hbm.at[pl.ds(r8, SUB)], dst_v)
            pltpu.sync_copy(src_hbm.at[pl.ds(r8, SUB)], src_v)
            gd = [None] * NBUF
            sd = [None] * NBUF
            gd[0] = pltpu.async_copy(x_hbm.at[src_v.at[0]], rows[0], gsem[0])
            for j in range(SUB):
                b = j % NBUF
                if j + 1 < SUB:
                    b2 = (j + 1) % NBUF
                    if sd[b2] is not None:
                        sd[b2].wait()
                    gd[b2] = pltpu.async_copy(
                        x_hbm.at[src_v.at[j + 1]], rows[b2], gsem[b2])
                gd[b].wait()
                sd[b] = pltpu.async_copy(
                    rows[b], acc_sh.at[dst_v.at[j]], ssem[b], add=True)
                for k in range(CHUNK // LANES):
                    dv = dst_v[j, pl.ds(LANES * k, LANES)]
                    plsc.addupdate_scatter(
                        hist_v,
                        [lax.shift_right_logical(dv, 7),
                         lax.bitwise_and(dv, 127)],
                        ones16)
            for b in range(NBUF):
                if sd[b] is not None:
                    sd[b].wait()
            return 0
        lax.fori_loop(0, LOADS_PER_TILE, load_step, 0)

    def copyout(out_hbm):
        pltpu.sync_copy(acc_sh.at[pl.ds(s * STRIPE, STRIPE)],
                        out_hbm.at[pl.ds(s * STRIPE, STRIPE)])

    def phase(fn1, fn2):
        pl.when(c == 0)(fn1)
        pl.when(c == 1)(fn2)
        plsc.subcore_barrier()

    def finish(acc_out, deg_out):
        copyout(acc_out)
        pltpu.sync_copy(hist_v, deg_out.at[s])

    phase(zero_acc, zero_acc)
    phase(lambda: scatter_pass(x1_hbm, d1_hbm),
          lambda: scatter_pass(x2_hbm, d2_hbm))
    phase(lambda: finish(acc1_out, deg1_out),
          lambda: finish(acc2_out, deg2_out))


def _sc_aggregate(src_pad, d1, d2, x1, x2):
    mesh = plsc.VectorSubcoreMesh(
        core_axis_name="c", subcore_axis_name="s",
        num_cores=NUM_CORES, num_subcores=NUM_SUBCORES)
    f32 = jnp.float32
    cp = pltpu.CompilerParams()
    if "needs_layout_passes" in pltpu.CompilerParams.__dataclass_fields__:
        cp = dataclasses.replace(cp, needs_layout_passes=False)
    kern = pl.kernel(
        _sc_body,
        out_type=(
            jax.ShapeDtypeStruct((ACC_ROWS, D_FEAT), f32),
            jax.ShapeDtypeStruct((NUM_SUBCORES, ACC_ROWS // 128, 128), f32),
            jax.ShapeDtypeStruct((ACC_ROWS, D_FEAT), f32),
            jax.ShapeDtypeStruct((NUM_SUBCORES, ACC_ROWS // 128, 128), f32),
        ),
        mesh=mesh,
        scratch_types=[
            pltpu.VMEM((SUB, 128), jnp.int32),
            pltpu.VMEM((SUB, 128), jnp.int32),
            pltpu.VMEM((CHUNK, D_FEAT), f32),
            pltpu.VMEM((CHUNK, D_FEAT), f32),
            pltpu.VMEM((ACC_ROWS // 128, 128), f32),
            pltpu.VMEM_SHARED((ACC_ROWS, D_FEAT), f32),
        ] + [pltpu.SemaphoreType.DMA] * 4,
        compiler_params=cp,
    )
    zeros_hbm = jnp.zeros((ACC_ROWS, D_FEAT), f32)
    return kern(src_pad, d1, d2, x1, x2, zeros_hbm)


def _epilogue_body(acc1_ref, deg1_ref, x1_ref, acc2_ref, deg2_ref, x2_ref,
                   w_ref, b_ref, z1_ref, z2_ref):
    w = w_ref[...]
    b = b_ref[...]
    blk = acc1_ref.shape[0]
    rows8 = blk // 128
    rsel = (lax.broadcasted_iota(jnp.int32, (blk, rows8), 0) // 128 ==
            lax.broadcasted_iota(jnp.int32, (blk, rows8), 1)).astype(
                jnp.float32)
    csel = (lax.broadcasted_iota(jnp.int32, (blk, 128), 0) % 128 ==
            lax.broadcasted_iota(jnp.int32, (blk, 128), 1)).astype(
                jnp.float32)

    def one(acc_ref, deg_ref, x_ref, z_ref):
        grid = jnp.sum(deg_ref[...], axis=0)
        t = jnp.dot(rsel, grid, preferred_element_type=jnp.float32,
                    precision=lax.Precision.HIGHEST)
        deg = jnp.sum(t * csel, axis=1, keepdims=True)
        deg = jnp.maximum(deg, 1.0)
        h = acc_ref[...] / deg + x_ref[...]
        z_ref[...] = jnp.dot(h, w, preferred_element_type=jnp.float32,
                             precision=lax.Precision.HIGHEST) + b

    one(acc1_ref, deg1_ref, x1_ref, z1_ref)
    one(acc2_ref, deg2_ref, x2_ref, z2_ref)


def _epilogue(acc1, deg1, x1, acc2, deg2, x2, W, b2d):
    blk = 1024
    grid = (ACC_ROWS // blk,)
    row_spec = pl.BlockSpec((blk, D_FEAT), lambda i: (i, 0))
    deg_spec = pl.BlockSpec((NUM_SUBCORES, blk // 128, 128),
                            lambda i: (0, i, 0))
    w_spec = pl.BlockSpec((D_FEAT, D_FEAT), lambda i: (0, 0))
    b_spec = pl.BlockSpec((1, D_FEAT), lambda i: (0, 0))
    return pl.pallas_call(
        _epilogue_body,
        grid=grid,
        in_specs=[row_spec, deg_spec, row_spec,
                  row_spec, deg_spec, row_spec, w_spec, b_spec],
        out_specs=(row_spec, row_spec),
        out_shape=(
            jax.ShapeDtypeStruct((ACC_ROWS, D_FEAT), jnp.float32),
            jax.ShapeDtypeStruct((ACC_ROWS, D_FEAT), jnp.float32),
        ),
    )(acc1, deg1, x1, acc2, deg2, x2, W, b2d)


@jax.jit
def kernel(x, edge_index, W, b):
    key = jax.random.key(42)
    k1, k2 = jax.random.split(key)
    ke1, kf1 = jax.random.split(k1)
    ke2, kf2 = jax.random.split(k2)
    ue1 = jax.random.uniform(ke1, (N_EDGES,))
    uf1 = jax.random.uniform(kf1, x.shape)
    ue2 = jax.random.uniform(ke2, (N_EDGES,))
    uf2 = jax.random.uniform(kf2, x.shape)

    src = edge_index[0]
    dst = edge_index[1]

    x1, x2, d1, d2 = _prologue(
        x, uf1, uf2,
        ue1.reshape(N_EDGES // 128, 128),
        ue2.reshape(N_EDGES // 128, 128),
        dst.reshape(N_EDGES // 128, 128),
    )

    src_pad = jnp.concatenate(
        [src, jnp.zeros((E_PAD - N_EDGES,), jnp.int32)]).reshape(
            E_PAD // 128, 128)
    acc1, deg1, acc2, deg2 = _sc_aggregate(src_pad, d1, d2, x1, x2)

    z1, z2 = _epilogue(acc1, deg1, x1, acc2, deg2, x2,
                       W, b.reshape(1, D_FEAT))
    return (z1[:N_NODES], z2[:N_NODES])
